# baseline (speedup 1.0000x reference)
import jax
import jax.numpy as jnp
from jax import lax
from jax.experimental import pallas as pl
from jax.experimental.pallas import tpu as pltpu
from jax.experimental.pallas import tpu_sc as plsc

NUM_CORES = 2
NUM_SUBCORES = 16
NUM_WORKERS = NUM_CORES * NUM_SUBCORES
LANES = 16

BATCH = 16384
HIDDEN = 128
ROWS_PER_WORKER = BATCH // NUM_WORKERS
CHUNK = 128
NUM_CHUNKS = ROWS_PER_WORKER // CHUNK
NBUF = 3
LAG = 2


def _mf_body(user_ids, item_ids, user_table, item_table, out_hbm,
             uidx, iidx, urows, irows, out_v,
             sem_uidx, sem_iidx, sem_urows, sem_irows):
  wid = lax.axis_index("s") * NUM_CORES + lax.axis_index("c")
  base = wid * ROWS_PER_WORKER

  pltpu.async_copy(user_ids.at[pl.ds(base, ROWS_PER_WORKER)], uidx, sem_uidx)
  pltpu.async_copy(item_ids.at[pl.ds(base, ROWS_PER_WORKER)], iidx, sem_iidx)

  HALF = CHUNK // 2

  def start_rows(c, p):
    for h in range(2):
      pltpu.async_copy(
          user_table.at[uidx.at[pl.ds(c * CHUNK + h * HALF, HALF)]],
          urows.at[pl.ds(p * CHUNK + h * HALF, HALF)], sem_urows.at[p])
      pltpu.async_copy(
          item_table.at[iidx.at[pl.ds(c * CHUNK + h * HALF, HALF)]],
          irows.at[pl.ds(p * CHUNK + h * HALF, HALF)], sem_irows.at[p])

  def wait_rows(c, p):
    for h in range(2):
      pltpu.make_async_copy(
          user_table.at[uidx.at[pl.ds(c * CHUNK + h * HALF, HALF)]],
          urows.at[pl.ds(p * CHUNK + h * HALF, HALF)], sem_urows.at[p]).wait()
      pltpu.make_async_copy(
          item_table.at[iidx.at[pl.ds(c * CHUNK + h * HALF, HALF)]],
          irows.at[pl.ds(p * CHUNK + h * HALF, HALF)], sem_irows.at[p]).wait()

  lane = lax.iota(jnp.int32, LANES)
  perms = [jnp.bitwise_xor(lane, k) for k in (8, 4, 2, 1)]
  mask0 = lane == 0

  def compute_chunk(cm):
    roff = lax.rem(cm, NBUF) * CHUNK

    @plsc.parallel_loop(0, CHUNK, unroll=4)
    def _(r):
      q = roff + r
      prods = [urows[q, pl.ds(j * LANES, LANES)] *
               irows[q, pl.ds(j * LANES, LANES)]
               for j in range(HIDDEN // LANES)]
      while len(prods) > 1:
        prods = [prods[i] + prods[i + 1] for i in range(0, len(prods), 2)]
      s = prods[0]
      for pm in perms:
        s = s + jnp.take_along_axis(s, pm, axis=0)
      rowid = jnp.full((LANES,), cm * CHUNK + r, jnp.int32)
      plsc.store_scatter(out_v, [rowid], s, mask=mask0)

  pltpu.make_async_copy(user_ids.at[pl.ds(base, ROWS_PER_WORKER)], uidx,
                        sem_uidx).wait()
  pltpu.make_async_copy(item_ids.at[pl.ds(base, ROWS_PER_WORKER)], iidx,
                        sem_iidx).wait()

  def pipe_body(c, _):
    ms = lax.rem(c, NBUF)
    mw = lax.rem(c - LAG + NBUF, NBUF)

    for b in range(NBUF):
      @pl.when(jnp.logical_and(c < NUM_CHUNKS, ms == b))
      def _(b=b):
        start_rows(c, b)

    for b in range(NBUF):
      @pl.when(jnp.logical_and(c >= LAG, mw == b))
      def _(b=b):
        wait_rows(c - LAG, b)

    @pl.when(c >= LAG)
    def _():
      compute_chunk(c - LAG)

    return 0

  lax.fori_loop(0, NUM_CHUNKS + LAG, pipe_body, 0)

  pltpu.sync_copy(out_v, out_hbm.at[pl.ds(base, ROWS_PER_WORKER)])


@jax.jit
def _mf_scores(batch_user_ids, batch_item_ids, user_table, item_table):
  mesh = plsc.VectorSubcoreMesh(
      core_axis_name="c", subcore_axis_name="s",
      num_cores=NUM_CORES, num_subcores=NUM_SUBCORES)
  grid_kernel = pl.kernel(
      _mf_body,
      out_type=jax.ShapeDtypeStruct((BATCH,), jnp.float32),
      mesh=mesh,
      compiler_params=pltpu.CompilerParams(needs_layout_passes=False),
      scratch_types=[
          pltpu.VMEM((ROWS_PER_WORKER,), jnp.int32),
          pltpu.VMEM((ROWS_PER_WORKER,), jnp.int32),
          pltpu.VMEM((NBUF * CHUNK, HIDDEN), jnp.float32),
          pltpu.VMEM((NBUF * CHUNK, HIDDEN), jnp.float32),
          pltpu.VMEM((ROWS_PER_WORKER,), jnp.float32),
          pltpu.SemaphoreType.DMA,
          pltpu.SemaphoreType.DMA,
          pltpu.SemaphoreType.DMA((NBUF,)),
          pltpu.SemaphoreType.DMA((NBUF,)),
      ],
  )
  return grid_kernel(batch_user_ids, batch_item_ids, user_table, item_table)


def kernel(batch_user_ids, batch_item_ids, user_table, item_table):
  return _mf_scores(batch_user_ids, batch_item_ids, user_table, item_table)


# speedup vs baseline: 1.0085x; 1.0085x over previous
import jax
import jax.numpy as jnp
from jax import lax
from jax.experimental import pallas as pl
from jax.experimental.pallas import tpu as pltpu
from jax.experimental.pallas import tpu_sc as plsc

NUM_CORES = 2
NUM_SUBCORES = 16
NUM_WORKERS = NUM_CORES * NUM_SUBCORES
LANES = 16

BATCH = 16384
HIDDEN = 128
ROWS_PER_WORKER = BATCH // NUM_WORKERS
CHUNK = 128
NUM_CHUNKS = ROWS_PER_WORKER // CHUNK
NBUF = 3
LAG = 2


def _mf_body(user_ids, item_ids, user_table, item_table, out_hbm,
             uidx, iidx, urows, irows, out_v,
             sem_uidx, sem_iidx, sem_urows, sem_irows):
  wid = lax.axis_index("s") * NUM_CORES + lax.axis_index("c")
  base = wid * ROWS_PER_WORKER

  pltpu.async_copy(user_ids.at[pl.ds(base, ROWS_PER_WORKER)], uidx, sem_uidx)
  pltpu.async_copy(item_ids.at[pl.ds(base, ROWS_PER_WORKER)], iidx, sem_iidx)

  def start_rows(c, p):
    pltpu.async_copy(user_table.at[uidx.at[pl.ds(c * CHUNK, CHUNK)]],
                     urows.at[pl.ds(p * CHUNK, CHUNK)], sem_urows.at[p])
    pltpu.async_copy(item_table.at[iidx.at[pl.ds(c * CHUNK, CHUNK)]],
                     irows.at[pl.ds(p * CHUNK, CHUNK)], sem_irows.at[p])

  def wait_rows(c, p):
    pltpu.make_async_copy(user_table.at[uidx.at[pl.ds(c * CHUNK, CHUNK)]],
                          urows.at[pl.ds(p * CHUNK, CHUNK)],
                          sem_urows.at[p]).wait()
    pltpu.make_async_copy(item_table.at[iidx.at[pl.ds(c * CHUNK, CHUNK)]],
                          irows.at[pl.ds(p * CHUNK, CHUNK)],
                          sem_irows.at[p]).wait()

  lane = lax.iota(jnp.int32, LANES)
  perms = [jnp.bitwise_xor(lane, k) for k in (8, 4, 2, 1)]
  mask0 = lane == 0

  def compute_chunk(cm):
    roff = lax.rem(cm, NBUF) * CHUNK

    @plsc.parallel_loop(0, CHUNK, unroll=2)
    def _(r):
      q = roff + r
      prods = [urows[q, pl.ds(j * LANES, LANES)] *
               irows[q, pl.ds(j * LANES, LANES)]
               for j in range(HIDDEN // LANES)]
      while len(prods) > 1:
        prods = [prods[i] + prods[i + 1] for i in range(0, len(prods), 2)]
      s = prods[0]
      for pm in perms:
        s = s + jnp.take_along_axis(s, pm, axis=0)
      rowid = jnp.full((LANES,), cm * CHUNK + r, jnp.int32)
      plsc.store_scatter(out_v, [rowid], s, mask=mask0)

  pltpu.make_async_copy(user_ids.at[pl.ds(base, ROWS_PER_WORKER)], uidx,
                        sem_uidx).wait()
  pltpu.make_async_copy(item_ids.at[pl.ds(base, ROWS_PER_WORKER)], iidx,
                        sem_iidx).wait()

  def pipe_body(c, _):
    ms = lax.rem(c, NBUF)
    mw = lax.rem(c - LAG + NBUF, NBUF)

    for b in range(NBUF):
      @pl.when(jnp.logical_and(c < NUM_CHUNKS, ms == b))
      def _(b=b):
        start_rows(c, b)

    for b in range(NBUF):
      @pl.when(jnp.logical_and(c >= LAG, mw == b))
      def _(b=b):
        wait_rows(c - LAG, b)

    @pl.when(c >= LAG)
    def _():
      compute_chunk(c - LAG)

    return 0

  lax.fori_loop(0, NUM_CHUNKS + LAG, pipe_body, 0)

  pltpu.sync_copy(out_v, out_hbm.at[pl.ds(base, ROWS_PER_WORKER)])


@jax.jit
def _mf_scores(batch_user_ids, batch_item_ids, user_table, item_table):
  mesh = plsc.VectorSubcoreMesh(
      core_axis_name="c", subcore_axis_name="s",
      num_cores=NUM_CORES, num_subcores=NUM_SUBCORES)
  grid_kernel = pl.kernel(
      _mf_body,
      out_type=jax.ShapeDtypeStruct((BATCH,), jnp.float32),
      mesh=mesh,
      compiler_params=pltpu.CompilerParams(needs_layout_passes=False),
      scratch_types=[
          pltpu.VMEM((ROWS_PER_WORKER,), jnp.int32),
          pltpu.VMEM((ROWS_PER_WORKER,), jnp.int32),
          pltpu.VMEM((NBUF * CHUNK, HIDDEN), jnp.float32),
          pltpu.VMEM((NBUF * CHUNK, HIDDEN), jnp.float32),
          pltpu.VMEM((ROWS_PER_WORKER,), jnp.float32),
          pltpu.SemaphoreType.DMA,
          pltpu.SemaphoreType.DMA,
          pltpu.SemaphoreType.DMA((NBUF,)),
          pltpu.SemaphoreType.DMA((NBUF,)),
      ],
  )
  return grid_kernel(batch_user_ids, batch_item_ids, user_table, item_table)


def kernel(batch_user_ids, batch_item_ids, user_table, item_table):
  return _mf_scores(batch_user_ids, batch_item_ids, user_table, item_table)
